# baseline (speedup 1.0000x reference)
import functools

import jax
import jax.numpy as jnp
from jax import lax
from jax.experimental import pallas as pl
from jax.experimental.pallas import tpu as pltpu
from jax.experimental.pallas import tpu_sc as plsc

SEQ_LEN = 2048
BATCH = 16
HIDDEN = 1024
HALF = HIDDEN // 2
NROWS = 2 * BATCH


def _body(table_hbm, lengths_hbm, out_hbm, len_v, idx_v, rows_v, sem):
    cid = lax.axis_index("c")
    sid = lax.axis_index("s")
    wid = sid * 2 + cid

    @pl.when(wid == 0)
    def _():
        pltpu.sync_copy(lengths_hbm, len_v)
        lens = len_v[...]
        b = lax.iota(jnp.int32, 16)
        fw = ((lens - 1) * BATCH + b) * 2
        bw = ((SEQ_LEN - lens) * BATCH + b) * 2 + 1
        plsc.store_scatter(idx_v, [b * 2], fw)
        plsc.store_scatter(idx_v, [b * 2 + 1], bw)
        pltpu.async_copy(table_hbm.at[idx_v], rows_v, sem).wait()
        pltpu.sync_copy(rows_v, out_hbm)


_gather = functools.partial(
    pl.kernel,
    out_type=jax.ShapeDtypeStruct((NROWS, HALF), jnp.float32),
    mesh=plsc.VectorSubcoreMesh(core_axis_name="c", subcore_axis_name="s"),
    scratch_types=[
        pltpu.VMEM((BATCH,), jnp.int32),
        pltpu.VMEM((NROWS,), jnp.int32),
        pltpu.VMEM((NROWS, HALF), jnp.float32),
        pltpu.SemaphoreType.DMA,
    ],
    compiler_params=pltpu.CompilerParams(needs_layout_passes=False),
)(_body)


@jax.jit
def kernel(lstm_out, lengths):
    table = lstm_out.reshape(SEQ_LEN * BATCH * 2, HALF)
    out = _gather(table, lengths.astype(jnp.int32))
    return out.reshape(BATCH, HIDDEN)


# speedup vs baseline: 6.9436x; 6.9436x over previous
import functools

import jax
import jax.numpy as jnp
from jax import lax
from jax.experimental import pallas as pl
from jax.experimental.pallas import tpu as pltpu
from jax.experimental.pallas import tpu_sc as plsc

SEQ_LEN = 2048
BATCH = 16
HIDDEN = 1024
HALF = HIDDEN // 2


def _body(table_hbm, lengths_hbm, out_hbm, len_v, idx_fw, idx_bw,
          rows_fw, rows_bw, sem_fw, sem_bw):
    cid = lax.axis_index("c")
    sid = lax.axis_index("s")
    wid = sid * 2 + cid

    @pl.when(wid == 0)
    def _():
        pltpu.sync_copy(lengths_hbm, len_v)
        lens = len_v[...]
        b = lax.iota(jnp.int32, 16)
        idx_fw[...] = (lens - 1) * BATCH + b
        idx_bw[...] = (SEQ_LEN - lens) * BATCH + b
        cp_fw = pltpu.async_copy(table_hbm.at[idx_fw], rows_fw, sem_fw)
        cp_bw = pltpu.async_copy(table_hbm.at[idx_bw], rows_bw, sem_bw)
        cp_fw.wait()
        cp_bw.wait()
        pltpu.sync_copy(rows_fw.at[:, pl.ds(0, HALF)],
                        out_hbm.at[:, pl.ds(0, HALF)])
        pltpu.sync_copy(rows_bw.at[:, pl.ds(HALF, HALF)],
                        out_hbm.at[:, pl.ds(HALF, HALF)])


_gather = functools.partial(
    pl.kernel,
    out_type=jax.ShapeDtypeStruct((BATCH, HIDDEN), jnp.float32),
    mesh=plsc.VectorSubcoreMesh(core_axis_name="c", subcore_axis_name="s"),
    scratch_types=[
        pltpu.VMEM((BATCH,), jnp.int32),
        pltpu.VMEM((BATCH,), jnp.int32),
        pltpu.VMEM((BATCH,), jnp.int32),
        pltpu.VMEM((BATCH, HIDDEN), jnp.float32),
        pltpu.VMEM((BATCH, HIDDEN), jnp.float32),
        pltpu.SemaphoreType.DMA,
        pltpu.SemaphoreType.DMA,
    ],
    compiler_params=pltpu.CompilerParams(needs_layout_passes=False),
)(_body)


@jax.jit
def kernel(lstm_out, lengths):
    table = lstm_out.reshape(SEQ_LEN * BATCH, HIDDEN)
    return _gather(table, lengths.astype(jnp.int32))


# speedup vs baseline: 7.4071x; 1.0668x over previous
import functools

import jax
import jax.numpy as jnp
from jax import lax
from jax.experimental import pallas as pl
from jax.experimental.pallas import tpu as pltpu
from jax.experimental.pallas import tpu_sc as plsc

SEQ_LEN = 2048
BATCH = 16
HIDDEN = 1024
HALF = HIDDEN // 2


def _body(table_hbm, lengths_hbm, out_hbm, len_v, idx_fw, idx_bw,
          rows_fw, rows_bw, sem_fw, sem_bw):
    cid = lax.axis_index("c")
    sid = lax.axis_index("s")
    wid = sid * 2 + cid

    @pl.when(wid == 0)
    def _():
        pltpu.sync_copy(lengths_hbm, len_v)
        lens = len_v[...]
        b = lax.iota(jnp.int32, 16)
        idx_fw[...] = (lens - 1) * BATCH + b
        idx_bw[...] = (SEQ_LEN - lens) * BATCH + b
        cp_fw = pltpu.async_copy(table_hbm.at[idx_fw], rows_fw, sem_fw)
        cp_bw = pltpu.async_copy(table_hbm.at[idx_bw], rows_bw, sem_bw)
        cp_fw.wait()
        cp_bw.wait()
        pltpu.sync_copy(rows_fw.at[:, pl.ds(0, HALF)],
                        out_hbm.at[:, pl.ds(0, HALF)])
        pltpu.sync_copy(rows_bw.at[:, pl.ds(HALF, HALF)],
                        out_hbm.at[:, pl.ds(HALF, HALF)])


_gather = functools.partial(
    pl.kernel,
    out_type=jax.ShapeDtypeStruct((BATCH, HIDDEN), jnp.float32),
    mesh=plsc.VectorSubcoreMesh(core_axis_name="c", subcore_axis_name="s",
                                num_cores=1),
    scratch_types=[
        pltpu.VMEM((BATCH,), jnp.int32),
        pltpu.VMEM((BATCH,), jnp.int32),
        pltpu.VMEM((BATCH,), jnp.int32),
        pltpu.VMEM((BATCH, HIDDEN), jnp.float32),
        pltpu.VMEM((BATCH, HIDDEN), jnp.float32),
        pltpu.SemaphoreType.DMA,
        pltpu.SemaphoreType.DMA,
    ],
    compiler_params=pltpu.CompilerParams(needs_layout_passes=False),
)(_body)


@jax.jit
def kernel(lstm_out, lengths):
    table = lstm_out.reshape(SEQ_LEN * BATCH, HIDDEN)
    return _gather(table, lengths.astype(jnp.int32))


# speedup vs baseline: 7.9421x; 1.0722x over previous
import functools

import jax
import jax.numpy as jnp
from jax import lax
from jax.experimental import pallas as pl
from jax.experimental.pallas import tpu as pltpu
from jax.experimental.pallas import tpu_sc as plsc

SEQ_LEN = 2048
BATCH = 16
HIDDEN = 1024
HALF = HIDDEN // 2


def _body(table_hbm, lengths_hbm, out_hbm, len_s, sem):
    pltpu.sync_copy(lengths_hbm, len_s)
    copies = []
    for b in range(BATCH):
        ln = len_s[b]
        copies.append(pltpu.async_copy(
            table_hbm.at[(ln - 1) * BATCH + b, pl.ds(0, HALF)],
            out_hbm.at[b, pl.ds(0, HALF)], sem))
        copies.append(pltpu.async_copy(
            table_hbm.at[(SEQ_LEN - ln) * BATCH + b, pl.ds(HALF, HALF)],
            out_hbm.at[b, pl.ds(HALF, HALF)], sem))
    for cp in copies:
        cp.wait()


_gather = functools.partial(
    pl.kernel,
    out_type=jax.ShapeDtypeStruct((BATCH, HIDDEN), jnp.float32),
    mesh=plsc.ScalarSubcoreMesh(axis_name="c", num_cores=1),
    scratch_types=[
        pltpu.SMEM((BATCH,), jnp.int32),
        pltpu.SemaphoreType.DMA,
    ],
    compiler_params=pltpu.CompilerParams(needs_layout_passes=False),
)(_body)


@jax.jit
def kernel(lstm_out, lengths):
    table = lstm_out.reshape(SEQ_LEN * BATCH, HIDDEN)
    return _gather(table, lengths.astype(jnp.int32))


# speedup vs baseline: 7.9769x; 1.0044x over previous
import functools

import jax
import jax.numpy as jnp
from jax import lax
from jax.experimental import pallas as pl
from jax.experimental.pallas import tpu as pltpu
from jax.experimental.pallas import tpu_sc as plsc

SEQ_LEN = 2048
BATCH = 16
HIDDEN = 1024
HALF = HIDDEN // 2


def _body(table_hbm, lengths_hbm, out_hbm, len_s, sem):
    pltpu.sync_copy(lengths_hbm, len_s)

    def issue(b, carry):
        ln = len_s[b]
        pltpu.async_copy(
            table_hbm.at[(ln - 1) * BATCH + b, pl.ds(0, HALF)],
            out_hbm.at[b, pl.ds(0, HALF)], sem)
        pltpu.async_copy(
            table_hbm.at[(SEQ_LEN - ln) * BATCH + b, pl.ds(HALF, HALF)],
            out_hbm.at[b, pl.ds(HALF, HALF)], sem)
        return carry

    lax.fori_loop(0, BATCH, issue, 0)
    pltpu.make_async_copy(table_hbm.at[pl.ds(0, BATCH), :], out_hbm, sem).wait()


_gather = functools.partial(
    pl.kernel,
    out_type=jax.ShapeDtypeStruct((BATCH, HIDDEN), jnp.float32),
    mesh=plsc.ScalarSubcoreMesh(axis_name="c", num_cores=1),
    scratch_types=[
        pltpu.SMEM((BATCH,), jnp.int32),
        pltpu.SemaphoreType.DMA,
    ],
    compiler_params=pltpu.CompilerParams(needs_layout_passes=False),
)(_body)


@jax.jit
def kernel(lstm_out, lengths):
    table = lstm_out.reshape(SEQ_LEN * BATCH, HIDDEN)
    return _gather(table, lengths.astype(jnp.int32))
